# baseline (speedup 1.0000x reference)
import jax
import jax.numpy as jnp
from jax import lax
from jax.experimental import pallas as pl
from jax.experimental.pallas import tpu as pltpu
from jax.experimental.pallas import tpu_sc as plsc

GRIDN = 128
TSZ = 16
NPTS = 1048576

NC = 2
NS = 16
NW = NC * NS
L = 16

PPW = NPTS // NW
C = 2048
NCHUNK = PPW // C
ROWS = C // 128
NPAIR = NCHUNK // 2 - 1

STRIDE_H = TSZ * GRIDN
STRIDE_D = GRIDN * TSZ * GRIDN


def _axis_index_weight(vals, s, o):
    ii = vals * s + o
    i0i = ii.astype(jnp.int32)
    w = ii - i0i.astype(jnp.float32)
    i0 = jnp.minimum(jnp.maximum(i0i, 0), GRIDN - 1)
    i1 = jnp.maximum(jnp.minimum(i0i + 1, GRIDN - 1), 0)
    return i0, i1, w


def _t_channel(tvals):
    v = (tvals + jnp.float32(1.0)) * jnp.float32(0.5) * jnp.float32(TSZ - 1)
    f = v.astype(jnp.int32)
    d = v - f.astype(jnp.float32)
    half = jnp.float32(0.5)
    bump = jnp.where(d > half, 1, jnp.where(d == half, f & 1, 0))
    r = f + bump
    return jnp.minimum(jnp.maximum(r, 0), TSZ - 1)


def _body(*refs):
    (vol_h, x_h, y_h, z_h, t_h, consts_h, out_h, consts_v) = refs[:8]
    bufA = refs[8:8 + 10]
    bufB = refs[18:18 + 10]
    in_semA, in_semB, gsemA, gsemB, out_semA, out_semB = refs[28:34]

    cid = lax.axis_index("c")
    sid = lax.axis_index("s")
    wid = sid * NC + cid
    base_w = wid * PPW

    pltpu.sync_copy(consts_h, consts_v)
    sx = consts_v[0]
    sy = consts_v[1]
    sz = consts_v[2]
    ox = consts_v[3]
    oy = consts_v[4]
    oz = consts_v[5]

    def unpack(buf):
        xv, yv, zv, tv, wxv, wyv, wzv = buf[0:7]
        idx = buf[7]
        vals = buf[8]
        outv = buf[9]
        return xv, yv, zv, tv, wxv, wyv, wzv, idx, vals, outv

    lanes = lax.broadcasted_iota(jnp.int32, (L,), 0)
    lanes2 = lanes * 2
    lanes2p1 = lanes2 + 1

    in_srcs = (x_h, y_h, z_h, t_h)

    def fire_in(c, buf, sem):
        base = base_w + c * C
        for src, dst in zip(in_srcs, buf[0:4]):
            pltpu.async_copy(src.at[pl.ds(base, C)], dst, sem)

    def wait_in(buf, sem):
        for src, dst in zip(in_srcs, buf[0:4]):
            pltpu.make_async_copy(src.at[pl.ds(0, C)], dst, sem).wait()

    def pass1(buf):
        xv, yv, zv, tv, wxv, wyv, wzv, idx, _, _ = unpack(buf)

        def row(r, carry):
            for k in range(8):
                sl = pl.ds(r * 128 + k * L, L)
                ix0, ix1, wx = _axis_index_weight(xv[sl], sx, ox)
                iy0, iy1, wy = _axis_index_weight(yv[sl], sy, oy)
                iz0, iz1, wz = _axis_index_weight(zv[sl], sz, oz)
                tt = _t_channel(tv[sl])
                wxv[sl] = wx
                wyv[sl] = wy
                wzv[sl] = wz
                tc = tt * GRIDN
                b00 = iz0 * STRIDE_D + iy0 * STRIDE_H + tc
                b01 = iz0 * STRIDE_D + iy1 * STRIDE_H + tc
                b10 = iz1 * STRIDE_D + iy0 * STRIDE_H + tc
                b11 = iz1 * STRIDE_D + iy1 * STRIDE_H + tc
                off2 = r * 256 + k * 2 * L
                psl0 = pl.ds(off2, 2 * L)
                psl1 = pl.ds(2 * C + off2, 2 * L)
                psl2 = pl.ds(4 * C + off2, 2 * L)
                psl3 = pl.ds(6 * C + off2, 2 * L)
                plsc.store_scatter(idx.at[psl0], [lanes2], b00 + ix0)
                plsc.store_scatter(idx.at[psl0], [lanes2p1], b00 + ix1)
                plsc.store_scatter(idx.at[psl1], [lanes2], b01 + ix0)
                plsc.store_scatter(idx.at[psl1], [lanes2p1], b01 + ix1)
                plsc.store_scatter(idx.at[psl2], [lanes2], b10 + ix0)
                plsc.store_scatter(idx.at[psl2], [lanes2p1], b10 + ix1)
                plsc.store_scatter(idx.at[psl3], [lanes2], b11 + ix0)
                plsc.store_scatter(idx.at[psl3], [lanes2p1], b11 + ix1)
            return carry

        lax.fori_loop(0, ROWS, row, 0)

    def fire_g(buf, gsem):
        _, _, _, _, _, _, _, idx, vals, _ = unpack(buf)
        pltpu.async_copy(vol_h.at[idx], vals, gsem)

    def wait_g(buf, gsem):
        _, _, _, _, _, _, _, idx, vals, _ = unpack(buf)
        pltpu.make_async_copy(vol_h.at[idx], vals, gsem).wait()

    def pass2(buf):
        _, _, _, _, wxv, wyv, wzv, _, vals, outv = unpack(buf)

        def row(r, carry):
            for k in range(8):
                sl = pl.ds(r * 128 + k * L, L)
                wx = wxv[sl]
                wy = wyv[sl]
                wz = wzv[sl]
                off2 = r * 256 + k * 2 * L
                psl0 = pl.ds(off2, 2 * L)
                psl1 = pl.ds(2 * C + off2, 2 * L)
                psl2 = pl.ds(4 * C + off2, 2 * L)
                psl3 = pl.ds(6 * C + off2, 2 * L)
                v000 = plsc.load_gather(vals.at[psl0], [lanes2])
                v001 = plsc.load_gather(vals.at[psl0], [lanes2p1])
                v010 = plsc.load_gather(vals.at[psl1], [lanes2])
                v011 = plsc.load_gather(vals.at[psl1], [lanes2p1])
                v100 = plsc.load_gather(vals.at[psl2], [lanes2])
                v101 = plsc.load_gather(vals.at[psl2], [lanes2p1])
                v110 = plsc.load_gather(vals.at[psl3], [lanes2])
                v111 = plsc.load_gather(vals.at[psl3], [lanes2p1])
                c00 = v000 + wx * (v001 - v000)
                c01 = v010 + wx * (v011 - v010)
                c10 = v100 + wx * (v101 - v100)
                c11 = v110 + wx * (v111 - v110)
                c0 = c00 + wy * (c01 - c00)
                c1 = c10 + wy * (c11 - c10)
                outv[sl] = c0 + wz * (c1 - c0)
            return carry

        lax.fori_loop(0, ROWS, row, 0)

    def fire_out(c, buf, sem):
        outv = buf[9]
        pltpu.async_copy(outv, out_h.at[pl.ds(base_w + c * C, C)], sem)

    def wait_out(buf, sem):
        outv = buf[9]
        pltpu.make_async_copy(outv, out_h.at[pl.ds(0, C)], sem).wait()

    last = NCHUNK - 1

    fire_in(0, bufA, in_semA)
    fire_in(1, bufB, in_semB)
    fire_out(0, bufA, out_semA)
    fire_out(1, bufB, out_semB)
    wait_in(bufA, in_semA)
    pass1(bufA)
    fire_g(bufA, gsemA)
    fire_in(2, bufA, in_semA)

    def pair(gi, carry):
        cb = 2 * gi + 1
        wait_in(bufB, in_semB)
        pass1(bufB)
        fire_g(bufB, gsemB)
        fire_in(jnp.minimum(cb + 2, last), bufB, in_semB)
        wait_g(bufA, gsemA)
        wait_out(bufA, out_semA)
        pass2(bufA)
        fire_out(2 * gi, bufA, out_semA)
        ca = 2 * gi + 2
        wait_in(bufA, in_semA)
        pass1(bufA)
        fire_g(bufA, gsemA)
        fire_in(jnp.minimum(ca + 2, last), bufA, in_semA)
        wait_g(bufB, gsemB)
        wait_out(bufB, out_semB)
        pass2(bufB)
        fire_out(cb, bufB, out_semB)
        return carry

    lax.fori_loop(0, NPAIR, pair, 0)

    wait_in(bufB, in_semB)
    pass1(bufB)
    fire_g(bufB, gsemB)
    wait_g(bufA, gsemA)
    wait_out(bufA, out_semA)
    pass2(bufA)
    fire_out(NCHUNK - 2, bufA, out_semA)
    wait_g(bufB, gsemB)
    wait_out(bufB, out_semB)
    pass2(bufB)
    fire_out(NCHUNK - 1, bufB, out_semB)
    wait_in(bufA, in_semA)
    wait_out(bufA, out_semA)
    wait_out(bufB, out_semB)


def _buf_types():
    return (
        [pltpu.VMEM((C,), jnp.float32) for _ in range(7)]
        + [pltpu.VMEM((8 * C,), jnp.int32)]
        + [pltpu.VMEM((8 * C,), jnp.float32)]
        + [pltpu.VMEM((C,), jnp.float32)]
    )


@jax.jit
def _run(vol_flat, x, y, z, t, consts):
    mesh = plsc.VectorSubcoreMesh(
        core_axis_name="c", subcore_axis_name="s", num_cores=NC, num_subcores=NS
    )
    f = pl.kernel(
        _body,
        out_type=jax.ShapeDtypeStruct((NPTS,), jnp.float32),
        mesh=mesh,
        compiler_params=pltpu.CompilerParams(needs_layout_passes=False),
        scratch_types=[pltpu.VMEM((8, L), jnp.float32)]
        + _buf_types()
        + _buf_types()
        + [pltpu.SemaphoreType.DMA] * 6,
    )
    return f(vol_flat, x, y, z, t, consts)


def kernel(xyz_sampled, t, aabb, alpha_volume):
    a0 = aabb[0]
    iv = jnp.float32(1.0) / (aabb[1] - aabb[0]) * jnp.float32(2.0)
    scale = iv * jnp.float32(0.5 * (GRIDN - 1))
    off = -a0 * scale
    consts = jnp.broadcast_to(
        jnp.concatenate([scale, off, jnp.zeros((2,), jnp.float32)])[:, None], (8, L)
    )
    vol_flat = jnp.transpose(alpha_volume, (0, 1, 3, 2)).reshape(-1)
    x = xyz_sampled[:, 0]
    y = xyz_sampled[:, 1]
    z = xyz_sampled[:, 2]
    return _run(vol_flat, x, y, z, t, consts)
